# baseline (speedup 1.0000x reference)
import functools

import jax
import jax.numpy as jnp
from jax import lax
from jax.experimental import pallas as pl
from jax.experimental.pallas import tpu as pltpu
from jax.experimental.pallas import tpu_sc as plsc

NC = 2
NS = 16
L = 16


def _hp_body(n, h_ref, w_ref, ws_ref, b_ref, e_ref, s_ref,
             hp_ref, sl_ref, g_ref, hs_ref):
    @pl.when(pl.program_id(1) == 0)
    def _selfloop():
        h = h_ref[...]
        hs_ref[...] = h.astype(jnp.bfloat16)
        sl_ref[...] = jnp.dot(h, ws_ref[...],
                              preferred_element_type=jnp.float32) + b_ref[...]

    hp_ref[...] = jnp.dot(hs_ref[...], w_ref[0],
                          preferred_element_type=jnp.float32)

    @pl.when((pl.program_id(0) == 0) & (pl.program_id(1) == 0))
    def _gidx():
        g_ref[...] = e_ref[...] * jnp.int32(n) + s_ref[...]


def _tc_transform(h, W, Ws, b, edges2d, src2d, bn):
    N, D = h.shape
    R = W.shape[0]
    nb = N // bn
    erows = edges2d.shape[0]
    return pl.pallas_call(
        functools.partial(_hp_body, N),
        grid=(nb, R),
        in_specs=[
            pl.BlockSpec((bn, D), lambda i, r: (i, 0)),
            pl.BlockSpec((1, D, D), lambda i, r: (r, 0, 0)),
            pl.BlockSpec((D, D), lambda i, r: (0, 0)),
            pl.BlockSpec((1, D), lambda i, r: (0, 0)),
            pl.BlockSpec((erows, 128), lambda i, r: (0, 0)),
            pl.BlockSpec((erows, 128), lambda i, r: (0, 0)),
        ],
        out_specs=[
            pl.BlockSpec((bn, D), lambda i, r: (r * nb + i, 0)),
            pl.BlockSpec((bn, D), lambda i, r: (i, 0)),
            pl.BlockSpec((erows, 128), lambda i, r: (0, 0)),
        ],
        out_shape=[
            jax.ShapeDtypeStruct((R * N, D), jnp.float32),
            jax.ShapeDtypeStruct((N, D), jnp.float32),
            jax.ShapeDtypeStruct((erows, 128), jnp.int32),
        ],
        scratch_shapes=[pltpu.VMEM((bn, D), jnp.bfloat16)],
    )(h, W, Ws, b, edges2d, src2d)


def _hpf_body(p_ref, sl_ref, w_ref, ws_ref, b_ref, hp_ref, slo_ref, hs_ref):
    @pl.when(pl.program_id(1) == 0)
    def _epilogue():
        h = jnp.maximum(p_ref[0] + p_ref[1] + sl_ref[...], 0.0)
        hs_ref[...] = h.astype(jnp.bfloat16)
        slo_ref[...] = jnp.dot(h, ws_ref[...],
                               preferred_element_type=jnp.float32) + b_ref[...]

    hp_ref[...] = jnp.dot(hs_ref[...], w_ref[0],
                          preferred_element_type=jnp.float32)


def _tc_fused(parts, sl, W, Ws, b, bn):
    _, N, D = parts.shape
    R = W.shape[0]
    nb = N // bn
    return pl.pallas_call(
        _hpf_body,
        grid=(nb, R),
        in_specs=[
            pl.BlockSpec((NC, bn, D), lambda i, r: (0, i, 0)),
            pl.BlockSpec((bn, D), lambda i, r: (i, 0)),
            pl.BlockSpec((1, D, D), lambda i, r: (r, 0, 0)),
            pl.BlockSpec((D, D), lambda i, r: (0, 0)),
            pl.BlockSpec((1, D), lambda i, r: (0, 0)),
        ],
        out_specs=[
            pl.BlockSpec((bn, D), lambda i, r: (r * nb + i, 0)),
            pl.BlockSpec((bn, D), lambda i, r: (i, 0)),
        ],
        out_shape=[
            jax.ShapeDtypeStruct((R * N, D), jnp.float32),
            jax.ShapeDtypeStruct((N, D), jnp.float32),
        ],
        scratch_shapes=[pltpu.VMEM((bn, D), jnp.bfloat16)],
    )(parts, sl, W, Ws, b)


def _fin_body(p_ref, sl_ref, o_ref):
    o_ref[...] = jnp.maximum(p_ref[0] + p_ref[1] + sl_ref[...], 0.0)


def _tc_finish(parts, sl, bn):
    _, N, D = parts.shape
    nb = N // bn
    return pl.pallas_call(
        _fin_body,
        grid=(nb,),
        in_specs=[
            pl.BlockSpec((NC, bn, D), lambda i: (0, i, 0)),
            pl.BlockSpec((bn, D), lambda i: (i, 0)),
        ],
        out_specs=pl.BlockSpec((bn, D), lambda i: (i, 0)),
        out_shape=jax.ShapeDtypeStruct((N, D), jnp.float32),
    )(parts, sl)


def _make_sc_scatter(N, D, E, R):
    ept = E // (NC * NS)
    ch = 96
    rem = ept % ch
    chunks = (ept - rem) // ch
    nbuf = 3
    rpt = (N // NS) // 8 * 8
    tail = N - rpt * NS
    assert ch % 8 == 0 and rem % 8 == 0 and tail % 8 == 0 and tail <= ch
    assert chunks >= nbuf

    mesh = plsc.VectorSubcoreMesh(core_axis_name="c", subcore_axis_name="s",
                                  num_cores=NC, num_subcores=NS)

    scratch = (
        [pltpu.VMEM((ept,), jnp.int32)]
        + [pltpu.VMEM((ch,), jnp.int32)] * nbuf
        + [pltpu.VMEM((ch, D), jnp.float32)] * nbuf
        + [pltpu.SemaphoreType.DMA] * (3 * nbuf)
        + [pltpu.VMEM((max(rem, 8),), jnp.int32),
           pltpu.VMEM((max(rem, 8), D), jnp.float32)]
        + [pltpu.VMEM_SHARED((N, D), jnp.float32)]
    )

    @functools.partial(
        pl.kernel,
        mesh=mesh,
        out_type=jax.ShapeDtypeStruct((NC, N, D), jnp.float32),
        scratch_types=scratch,
    )
    def sc_scatter(hp, gidx, dst, out, *refs):
        gidxs = refs[0]
        dbufs = refs[1:1 + nbuf]
        rbufs = refs[1 + nbuf:1 + 2 * nbuf]
        gsems = refs[1 + 2 * nbuf:1 + 3 * nbuf]
        dsems = refs[1 + 3 * nbuf:1 + 4 * nbuf]
        ssems = refs[1 + 4 * nbuf:1 + 5 * nbuf]
        dbuf_rem = refs[1 + 5 * nbuf]
        rbuf_rem = refs[2 + 5 * nbuf]
        acc = refs[3 + 5 * nbuf]

        c = lax.axis_index("c")
        s = lax.axis_index("s")

        zv = jnp.zeros((L,), jnp.float32)

        def zero_row(i, carry):
            for j in range(D // L):
                rbufs[0][i, pl.ds(j * L, L)] = zv
            return carry

        lax.fori_loop(0, ch, zero_row, 0)
        for k in range(rpt // ch):
            pltpu.sync_copy(rbufs[0], acc.at[pl.ds(s * rpt + k * ch, ch)])
        zrem = rpt % ch
        if zrem:
            pltpu.sync_copy(rbufs[0].at[pl.ds(0, zrem)],
                            acc.at[pl.ds(s * rpt + (rpt // ch) * ch, zrem)])

        @pl.when(s == NS - 1)
        def _zero_tail():
            pltpu.sync_copy(rbufs[0].at[pl.ds(0, tail)],
                            acc.at[pl.ds(NS * rpt, tail)])

        plsc.subcore_barrier()

        base = (c * NS + s) * ept
        pltpu.sync_copy(gidx.at[pl.ds(base, ept)], gidxs)

        if rem:
            pltpu.sync_copy(dst.at[pl.ds(base, rem)], dbuf_rem)
            pltpu.sync_copy(hp.at[gidxs.at[pl.ds(0, rem)]], rbuf_rem)
            pltpu.sync_copy(rbuf_rem, acc.at[dbuf_rem], add=True)

        def start_fetch(chunk_idx, b):
            off = rem + chunk_idx * ch
            pltpu.async_copy(dst.at[pl.ds(base + off, ch)], dbufs[b],
                             dsems[b])
            pltpu.async_copy(hp.at[gidxs.at[pl.ds(off, ch)]], rbufs[b],
                             gsems[b])

        for b in range(nbuf):
            start_fetch(jnp.int32(b), b)

        def body(k, carry):
            i0 = k * nbuf
            for b in range(nbuf):
                cidx = i0 + b

                @pl.when(cidx < chunks)
                def _drain():
                    pltpu.make_async_copy(hp.at[gidxs.at[pl.ds(0, ch)]],
                                          rbufs[b], gsems[b]).wait()
                    pltpu.make_async_copy(dst.at[pl.ds(base, ch)],
                                          dbufs[b], dsems[b]).wait()
                    pltpu.async_copy(rbufs[b], acc.at[dbufs[b]], ssems[b],
                                     add=True)

                @pl.when(cidx + nbuf < chunks)
                def _refill():
                    pltpu.make_async_copy(rbufs[b], acc.at[dbufs[b]],
                                          ssems[b]).wait()
                    start_fetch(cidx + nbuf, b)
            return carry

        lax.fori_loop(0, (chunks + nbuf - 1) // nbuf, body, 0)

        for b in range(nbuf):
            pltpu.make_async_copy(rbufs[b], acc.at[dbufs[b]],
                                  ssems[b]).wait()
        plsc.subcore_barrier()

        r0 = s * rpt
        pltpu.sync_copy(acc.at[pl.ds(r0, rpt)], out.at[c, pl.ds(r0, rpt)])

        @pl.when(s == NS - 1)
        def _copy_tail():
            pltpu.sync_copy(acc.at[pl.ds(NS * rpt, tail)],
                            out.at[c, pl.ds(NS * rpt, tail)])

    return sc_scatter


def kernel(g, feat, edges, W0, Ws0, b0, W1, Ws1, b1, W2, Ws2, b2):
    dst = g[1]
    N, D = feat.shape
    E = edges.shape[0]
    R = W0.shape[0]
    bn = 2000

    sc_scatter = _make_sc_scatter(N, D, E, R)

    hp, sl, gidx2d = _tc_transform(feat, W0.astype(jnp.bfloat16), Ws0,
                                   b0.reshape(1, D), edges.reshape(-1, 128),
                                   g[0].reshape(-1, 128), bn)
    gidx = gidx2d.reshape(E)
    parts = sc_scatter(hp, gidx, dst)
    for (W, Ws, b) in ((W1, Ws1, b1), (W2, Ws2, b2)):
        hp, sl = _tc_fused(parts, sl, W.astype(jnp.bfloat16), Ws,
                           b.reshape(1, D), bn)
        parts = sc_scatter(hp, gidx, dst)
    return _tc_finish(parts, sl, bn)


# speedup vs baseline: 1.1735x; 1.1735x over previous
import functools

import jax
import jax.numpy as jnp
from jax import lax
from jax.experimental import pallas as pl
from jax.experimental.pallas import tpu as pltpu
from jax.experimental.pallas import tpu_sc as plsc

NC = 2
NS = 16
L = 16


def _hp_body(n, h_ref, w_ref, ws_ref, b_ref, e_ref, s_ref,
             hp_ref, sl_ref, g_ref, hs_ref):
    @pl.when(pl.program_id(1) == 0)
    def _selfloop():
        h = h_ref[...]
        hs_ref[...] = h
        sl_ref[...] = jnp.dot(h, ws_ref[...],
                              preferred_element_type=jnp.float32) + b_ref[...]

    hp_ref[...] = jnp.dot(hs_ref[...], w_ref[0],
                          preferred_element_type=jnp.float32)

    @pl.when((pl.program_id(0) == 0) & (pl.program_id(1) == 0))
    def _gidx():
        g_ref[...] = e_ref[...] * jnp.int32(n) + s_ref[...]


def _tc_transform(h, W, Ws, b, edges2d, src2d, bn):
    N, D = h.shape
    R = W.shape[0]
    nb = N // bn
    erows = edges2d.shape[0]
    return pl.pallas_call(
        functools.partial(_hp_body, N),
        grid=(nb, R),
        in_specs=[
            pl.BlockSpec((bn, D), lambda i, r: (i, 0)),
            pl.BlockSpec((1, D, D), lambda i, r: (r, 0, 0)),
            pl.BlockSpec((D, D), lambda i, r: (0, 0)),
            pl.BlockSpec((1, D), lambda i, r: (0, 0)),
            pl.BlockSpec((erows, 128), lambda i, r: (0, 0)),
            pl.BlockSpec((erows, 128), lambda i, r: (0, 0)),
        ],
        out_specs=[
            pl.BlockSpec((bn, D), lambda i, r: (r * nb + i, 0)),
            pl.BlockSpec((bn, D), lambda i, r: (i, 0)),
            pl.BlockSpec((erows, 128), lambda i, r: (0, 0)),
        ],
        out_shape=[
            jax.ShapeDtypeStruct((R * N, D), jnp.float32),
            jax.ShapeDtypeStruct((N, D), jnp.float32),
            jax.ShapeDtypeStruct((erows, 128), jnp.int32),
        ],
        scratch_shapes=[pltpu.VMEM((bn, D), jnp.float32)],
    )(h, W, Ws, b, edges2d, src2d)


def _hpf_body(p_ref, sl_ref, w_ref, ws_ref, b_ref, hp_ref, slo_ref, hs_ref):
    @pl.when(pl.program_id(1) == 0)
    def _epilogue():
        h = jnp.maximum(p_ref[0] + p_ref[1] + sl_ref[...], 0.0)
        hs_ref[...] = h
        slo_ref[...] = jnp.dot(h, ws_ref[...],
                               preferred_element_type=jnp.float32) + b_ref[...]

    hp_ref[...] = jnp.dot(hs_ref[...], w_ref[0],
                          preferred_element_type=jnp.float32)


def _tc_fused(parts, sl, W, Ws, b, bn):
    _, N, D = parts.shape
    R = W.shape[0]
    nb = N // bn
    return pl.pallas_call(
        _hpf_body,
        grid=(nb, R),
        in_specs=[
            pl.BlockSpec((NC, bn, D), lambda i, r: (0, i, 0)),
            pl.BlockSpec((bn, D), lambda i, r: (i, 0)),
            pl.BlockSpec((1, D, D), lambda i, r: (r, 0, 0)),
            pl.BlockSpec((D, D), lambda i, r: (0, 0)),
            pl.BlockSpec((1, D), lambda i, r: (0, 0)),
        ],
        out_specs=[
            pl.BlockSpec((bn, D), lambda i, r: (r * nb + i, 0)),
            pl.BlockSpec((bn, D), lambda i, r: (i, 0)),
        ],
        out_shape=[
            jax.ShapeDtypeStruct((R * N, D), jnp.float32),
            jax.ShapeDtypeStruct((N, D), jnp.float32),
        ],
        scratch_shapes=[pltpu.VMEM((bn, D), jnp.float32)],
    )(parts, sl, W, Ws, b)


def _fin_body(p_ref, sl_ref, o_ref):
    o_ref[...] = jnp.maximum(p_ref[0] + p_ref[1] + sl_ref[...], 0.0)


def _tc_finish(parts, sl, bn):
    _, N, D = parts.shape
    nb = N // bn
    return pl.pallas_call(
        _fin_body,
        grid=(nb,),
        in_specs=[
            pl.BlockSpec((NC, bn, D), lambda i: (0, i, 0)),
            pl.BlockSpec((bn, D), lambda i: (i, 0)),
        ],
        out_specs=pl.BlockSpec((bn, D), lambda i: (i, 0)),
        out_shape=jax.ShapeDtypeStruct((N, D), jnp.float32),
    )(parts, sl)


def _make_sc_scatter(N, D, E, R):
    ept = E // (NC * NS)
    ch = 96
    rem = ept % ch
    chunks = (ept - rem) // ch
    nbuf = 3
    rpt = (N // NS) // 8 * 8
    tail = N - rpt * NS
    assert ch % 8 == 0 and rem % 8 == 0 and tail % 8 == 0 and tail <= ch
    assert chunks >= nbuf

    mesh = plsc.VectorSubcoreMesh(core_axis_name="c", subcore_axis_name="s",
                                  num_cores=NC, num_subcores=NS)

    scratch = (
        [pltpu.VMEM((ept,), jnp.int32)]
        + [pltpu.VMEM((ch,), jnp.int32)] * nbuf
        + [pltpu.VMEM((ch, D), jnp.float32)] * nbuf
        + [pltpu.SemaphoreType.DMA] * (3 * nbuf)
        + [pltpu.VMEM((max(rem, 8),), jnp.int32),
           pltpu.VMEM((max(rem, 8), D), jnp.float32)]
        + [pltpu.VMEM_SHARED((N, D), jnp.float32)]
    )

    @functools.partial(
        pl.kernel,
        mesh=mesh,
        out_type=jax.ShapeDtypeStruct((NC, N, D), jnp.float32),
        scratch_types=scratch,
    )
    def sc_scatter(hp, gidx, dst, out, *refs):
        gidxs = refs[0]
        dbufs = refs[1:1 + nbuf]
        rbufs = refs[1 + nbuf:1 + 2 * nbuf]
        gsems = refs[1 + 2 * nbuf:1 + 3 * nbuf]
        dsems = refs[1 + 3 * nbuf:1 + 4 * nbuf]
        ssems = refs[1 + 4 * nbuf:1 + 5 * nbuf]
        dbuf_rem = refs[1 + 5 * nbuf]
        rbuf_rem = refs[2 + 5 * nbuf]
        acc = refs[3 + 5 * nbuf]

        c = lax.axis_index("c")
        s = lax.axis_index("s")

        zv = jnp.zeros((L,), jnp.float32)

        def zero_row(i, carry):
            for j in range(D // L):
                rbufs[0][i, pl.ds(j * L, L)] = zv
            return carry

        lax.fori_loop(0, ch, zero_row, 0)
        for k in range(rpt // ch):
            pltpu.sync_copy(rbufs[0], acc.at[pl.ds(s * rpt + k * ch, ch)])
        zrem = rpt % ch
        if zrem:
            pltpu.sync_copy(rbufs[0].at[pl.ds(0, zrem)],
                            acc.at[pl.ds(s * rpt + (rpt // ch) * ch, zrem)])

        @pl.when(s == NS - 1)
        def _zero_tail():
            pltpu.sync_copy(rbufs[0].at[pl.ds(0, tail)],
                            acc.at[pl.ds(NS * rpt, tail)])

        plsc.subcore_barrier()

        base = (c * NS + s) * ept
        pltpu.sync_copy(gidx.at[pl.ds(base, ept)], gidxs)

        if rem:
            pltpu.sync_copy(dst.at[pl.ds(base, rem)], dbuf_rem)
            pltpu.sync_copy(hp.at[gidxs.at[pl.ds(0, rem)]], rbuf_rem)
            pltpu.sync_copy(rbuf_rem, acc.at[dbuf_rem], add=True)

        def start_fetch(chunk_idx, b):
            off = rem + chunk_idx * ch
            pltpu.async_copy(dst.at[pl.ds(base + off, ch)], dbufs[b],
                             dsems[b])
            pltpu.async_copy(hp.at[gidxs.at[pl.ds(off, ch)]], rbufs[b],
                             gsems[b])

        for b in range(nbuf):
            start_fetch(jnp.int32(b), b)

        def body(k, carry):
            i0 = k * nbuf
            for b in range(nbuf):
                cidx = i0 + b

                @pl.when(cidx < chunks)
                def _drain():
                    pltpu.make_async_copy(hp.at[gidxs.at[pl.ds(0, ch)]],
                                          rbufs[b], gsems[b]).wait()
                    pltpu.make_async_copy(dst.at[pl.ds(base, ch)],
                                          dbufs[b], dsems[b]).wait()
                    pltpu.async_copy(rbufs[b], acc.at[dbufs[b]], ssems[b],
                                     add=True)

                @pl.when(cidx + nbuf < chunks)
                def _refill():
                    pltpu.make_async_copy(rbufs[b], acc.at[dbufs[b]],
                                          ssems[b]).wait()
                    start_fetch(cidx + nbuf, b)
            return carry

        lax.fori_loop(0, (chunks + nbuf - 1) // nbuf, body, 0)

        for b in range(nbuf):
            pltpu.make_async_copy(rbufs[b], acc.at[dbufs[b]],
                                  ssems[b]).wait()
        plsc.subcore_barrier()

        r0 = s * rpt
        pltpu.sync_copy(acc.at[pl.ds(r0, rpt)], out.at[c, pl.ds(r0, rpt)])

        @pl.when(s == NS - 1)
        def _copy_tail():
            pltpu.sync_copy(acc.at[pl.ds(NS * rpt, tail)],
                            out.at[c, pl.ds(NS * rpt, tail)])

    return sc_scatter


def kernel(g, feat, edges, W0, Ws0, b0, W1, Ws1, b1, W2, Ws2, b2):
    dst = g[1]
    N, D = feat.shape
    E = edges.shape[0]
    R = W0.shape[0]
    bn = 10000

    sc_scatter = _make_sc_scatter(N, D, E, R)

    hp, sl, gidx2d = _tc_transform(feat, W0, Ws0, b0.reshape(1, D),
                                   edges.reshape(-1, 128),
                                   g[0].reshape(-1, 128), bn)
    gidx = gidx2d.reshape(E)
    parts = sc_scatter(hp, gidx, dst)
    for (W, Ws, b) in ((W1, Ws1, b1), (W2, Ws2, b2)):
        hp, sl = _tc_fused(parts, sl, W, Ws, b.reshape(1, D), bn)
        parts = sc_scatter(hp, gidx, dst)
    return _tc_finish(parts, sl, bn)


# speedup vs baseline: 1.1773x; 1.0032x over previous
import functools

import jax
import jax.numpy as jnp
from jax import lax
from jax.experimental import pallas as pl
from jax.experimental.pallas import tpu as pltpu
from jax.experimental.pallas import tpu_sc as plsc

NC = 2
NS = 16
L = 16


def _hp_body(n, h_ref, w_ref, ws_ref, b_ref, e_ref, s_ref,
             hp_ref, sl_ref, g_ref, hs_ref):
    @pl.when(pl.program_id(1) == 0)
    def _selfloop():
        h = h_ref[...]
        hs_ref[...] = h
        sl_ref[...] = jnp.dot(h, ws_ref[...],
                              preferred_element_type=jnp.float32) + b_ref[...]

    hp_ref[...] = jnp.dot(hs_ref[...], w_ref[0],
                          preferred_element_type=jnp.float32)

    @pl.when((pl.program_id(0) == 0) & (pl.program_id(1) == 0))
    def _gidx():
        g_ref[...] = e_ref[...] * jnp.int32(n) + s_ref[...]


def _tc_transform(h, W, Ws, b, edges2d, src2d, bn):
    N, D = h.shape
    R = W.shape[0]
    nb = N // bn
    erows = edges2d.shape[0]
    return pl.pallas_call(
        functools.partial(_hp_body, N),
        grid=(nb, R),
        in_specs=[
            pl.BlockSpec((bn, D), lambda i, r: (i, 0)),
            pl.BlockSpec((1, D, D), lambda i, r: (r, 0, 0)),
            pl.BlockSpec((D, D), lambda i, r: (0, 0)),
            pl.BlockSpec((1, D), lambda i, r: (0, 0)),
            pl.BlockSpec((erows, 128), lambda i, r: (0, 0)),
            pl.BlockSpec((erows, 128), lambda i, r: (0, 0)),
        ],
        out_specs=[
            pl.BlockSpec((bn, D), lambda i, r: (r * nb + i, 0)),
            pl.BlockSpec((bn, D), lambda i, r: (i, 0)),
            pl.BlockSpec((erows, 128), lambda i, r: (0, 0)),
        ],
        out_shape=[
            jax.ShapeDtypeStruct((R * N, D), jnp.float32),
            jax.ShapeDtypeStruct((N, D), jnp.float32),
            jax.ShapeDtypeStruct((erows, 128), jnp.int32),
        ],
        scratch_shapes=[pltpu.VMEM((bn, D), jnp.float32)],
    )(h, W, Ws, b, edges2d, src2d)


def _hpf_body(p_ref, sl_ref, w_ref, ws_ref, b_ref, hp_ref, slo_ref, hs_ref):
    @pl.when(pl.program_id(1) == 0)
    def _epilogue():
        h = jnp.maximum(p_ref[0] + p_ref[1] + sl_ref[...], 0.0)
        hs_ref[...] = h
        slo_ref[...] = jnp.dot(h, ws_ref[...],
                               preferred_element_type=jnp.float32) + b_ref[...]

    hp_ref[...] = jnp.dot(hs_ref[...], w_ref[0],
                          preferred_element_type=jnp.float32)


def _tc_fused(parts, sl, W, Ws, b, bn):
    _, N, D = parts.shape
    R = W.shape[0]
    nb = N // bn
    return pl.pallas_call(
        _hpf_body,
        grid=(nb, R),
        in_specs=[
            pl.BlockSpec((NC, bn, D), lambda i, r: (0, i, 0)),
            pl.BlockSpec((bn, D), lambda i, r: (i, 0)),
            pl.BlockSpec((1, D, D), lambda i, r: (r, 0, 0)),
            pl.BlockSpec((D, D), lambda i, r: (0, 0)),
            pl.BlockSpec((1, D), lambda i, r: (0, 0)),
        ],
        out_specs=[
            pl.BlockSpec((bn, D), lambda i, r: (r * nb + i, 0)),
            pl.BlockSpec((bn, D), lambda i, r: (i, 0)),
        ],
        out_shape=[
            jax.ShapeDtypeStruct((R * N, D), jnp.float32),
            jax.ShapeDtypeStruct((N, D), jnp.float32),
        ],
        scratch_shapes=[pltpu.VMEM((bn, D), jnp.float32)],
    )(parts, sl, W, Ws, b)


def _fin_body(p_ref, sl_ref, o_ref):
    o_ref[...] = jnp.maximum(p_ref[0] + p_ref[1] + sl_ref[...], 0.0)


def _tc_finish(parts, sl, bn):
    _, N, D = parts.shape
    nb = N // bn
    return pl.pallas_call(
        _fin_body,
        grid=(nb,),
        in_specs=[
            pl.BlockSpec((NC, bn, D), lambda i: (0, i, 0)),
            pl.BlockSpec((bn, D), lambda i: (i, 0)),
        ],
        out_specs=pl.BlockSpec((bn, D), lambda i: (i, 0)),
        out_shape=jax.ShapeDtypeStruct((N, D), jnp.float32),
    )(parts, sl)


def _make_sc_scatter(N, D, E, R):
    ept = E // (NC * NS)
    ch = 104
    rem = ept % ch
    chunks = (ept - rem) // ch
    nbuf = 3
    rpt = (N // NS) // 8 * 8
    tail = N - rpt * NS
    assert ch % 8 == 0 and rem % 8 == 0 and tail % 8 == 0 and tail <= ch
    assert chunks >= nbuf

    mesh = plsc.VectorSubcoreMesh(core_axis_name="c", subcore_axis_name="s",
                                  num_cores=NC, num_subcores=NS)

    scratch = (
        [pltpu.VMEM((ept,), jnp.int32)]
        + [pltpu.VMEM((ch,), jnp.int32)] * nbuf
        + [pltpu.VMEM((ch, D), jnp.float32)] * nbuf
        + [pltpu.SemaphoreType.DMA] * (3 * nbuf)
        + [pltpu.VMEM((max(rem, 8),), jnp.int32)]
        + [pltpu.VMEM_SHARED((N, D), jnp.float32)]
    )

    @functools.partial(
        pl.kernel,
        mesh=mesh,
        out_type=jax.ShapeDtypeStruct((NC, N, D), jnp.float32),
        scratch_types=scratch,
    )
    def sc_scatter(hp, gidx, dst, out, *refs):
        gidxs = refs[0]
        dbufs = refs[1:1 + nbuf]
        rbufs = refs[1 + nbuf:1 + 2 * nbuf]
        gsems = refs[1 + 2 * nbuf:1 + 3 * nbuf]
        dsems = refs[1 + 3 * nbuf:1 + 4 * nbuf]
        ssems = refs[1 + 4 * nbuf:1 + 5 * nbuf]
        dbuf_rem = refs[1 + 5 * nbuf]
        acc = refs[2 + 5 * nbuf]

        c = lax.axis_index("c")
        s = lax.axis_index("s")

        zv = jnp.zeros((L,), jnp.float32)

        def zero_row(i, carry):
            for j in range(D // L):
                rbufs[0][i, pl.ds(j * L, L)] = zv
            return carry

        lax.fori_loop(0, ch, zero_row, 0)
        for k in range(rpt // ch):
            pltpu.sync_copy(rbufs[0], acc.at[pl.ds(s * rpt + k * ch, ch)])
        zrem = rpt % ch
        if zrem:
            pltpu.sync_copy(rbufs[0].at[pl.ds(0, zrem)],
                            acc.at[pl.ds(s * rpt + (rpt // ch) * ch, zrem)])

        @pl.when(s == NS - 1)
        def _zero_tail():
            pltpu.sync_copy(rbufs[0].at[pl.ds(0, tail)],
                            acc.at[pl.ds(NS * rpt, tail)])

        plsc.subcore_barrier()

        base = (c * NS + s) * ept
        pltpu.sync_copy(gidx.at[pl.ds(base, ept)], gidxs)

        if rem:
            pltpu.sync_copy(dst.at[pl.ds(base, rem)], dbuf_rem)
            pltpu.sync_copy(hp.at[gidxs.at[pl.ds(0, rem)]],
                            rbufs[0].at[pl.ds(0, rem)])
            pltpu.sync_copy(rbufs[0].at[pl.ds(0, rem)], acc.at[dbuf_rem],
                            add=True)

        def start_fetch(chunk_idx, b):
            off = rem + chunk_idx * ch
            pltpu.async_copy(dst.at[pl.ds(base + off, ch)], dbufs[b],
                             dsems[b])
            pltpu.async_copy(hp.at[gidxs.at[pl.ds(off, ch)]], rbufs[b],
                             gsems[b])

        for b in range(nbuf):
            start_fetch(jnp.int32(b), b)

        def body(k, carry):
            i0 = k * nbuf
            for b in range(nbuf):
                cidx = i0 + b

                @pl.when(cidx < chunks)
                def _drain():
                    pltpu.make_async_copy(hp.at[gidxs.at[pl.ds(0, ch)]],
                                          rbufs[b], gsems[b]).wait()
                    pltpu.make_async_copy(dst.at[pl.ds(base, ch)],
                                          dbufs[b], dsems[b]).wait()
                    pltpu.async_copy(rbufs[b], acc.at[dbufs[b]], ssems[b],
                                     add=True)

                @pl.when(cidx + nbuf < chunks)
                def _refill():
                    pltpu.make_async_copy(rbufs[b], acc.at[dbufs[b]],
                                          ssems[b]).wait()
                    start_fetch(cidx + nbuf, b)
            return carry

        lax.fori_loop(0, (chunks + nbuf - 1) // nbuf, body, 0)

        for b in range(nbuf):
            pltpu.make_async_copy(rbufs[b], acc.at[dbufs[b]],
                                  ssems[b]).wait()
        plsc.subcore_barrier()

        r0 = s * rpt
        pltpu.sync_copy(acc.at[pl.ds(r0, rpt)], out.at[c, pl.ds(r0, rpt)])

        @pl.when(s == NS - 1)
        def _copy_tail():
            pltpu.sync_copy(acc.at[pl.ds(NS * rpt, tail)],
                            out.at[c, pl.ds(NS * rpt, tail)])

    return sc_scatter


def kernel(g, feat, edges, W0, Ws0, b0, W1, Ws1, b1, W2, Ws2, b2):
    dst = g[1]
    N, D = feat.shape
    E = edges.shape[0]
    R = W0.shape[0]
    bn = 10000

    sc_scatter = _make_sc_scatter(N, D, E, R)

    hp, sl, gidx2d = _tc_transform(feat, W0, Ws0, b0.reshape(1, D),
                                   edges.reshape(-1, 128),
                                   g[0].reshape(-1, 128), bn)
    gidx = gidx2d.reshape(E)
    parts = sc_scatter(hp, gidx, dst)
    for (W, Ws, b) in ((W1, Ws1, b1), (W2, Ws2, b2)):
        hp, sl = _tc_fused(parts, sl, W, Ws, b.reshape(1, D), bn)
        parts = sc_scatter(hp, gidx, dst)
    return _tc_finish(parts, sl, bn)
